# baseline (speedup 1.0000x reference)
import functools

import jax
import jax.numpy as jnp
from jax import lax
from jax.experimental import pallas as pl
from jax.experimental.pallas import tpu as pltpu
from jax.experimental.pallas import tpu_sc as plsc


def _gather_rows(table, token_ids, ntok, d):
    info = plsc.get_sparse_core_info()
    nw = info.num_cores * info.num_subcores
    b_per_w = ntok // nw
    chunk = min(64, b_per_w)
    nchunks = b_per_w // chunk

    mesh = plsc.VectorSubcoreMesh(core_axis_name="c", subcore_axis_name="s")

    @functools.partial(
        pl.kernel,
        mesh=mesh,
        out_type=jax.ShapeDtypeStruct((ntok, d), jnp.float32),
        scratch_types=[
            pltpu.VMEM((b_per_w,), jnp.int32),
            pltpu.VMEM((chunk, d), jnp.float32),
            pltpu.VMEM((chunk, d), jnp.float32),
            pltpu.SemaphoreType.DMA,
            pltpu.SemaphoreType.DMA,
        ],
    )
    def gather_kernel(table_hbm, ids_hbm, out_hbm, idx_v, rows0, rows1, sem0, sem1):
        wid = lax.axis_index("s") * info.num_cores + lax.axis_index("c")
        base = wid * b_per_w
        pltpu.sync_copy(ids_hbm.at[pl.ds(base, b_per_w)], idx_v)
        bufs = (rows0, rows1)
        sems = (sem0, sem1)
        copies = [None] * nchunks
        copies[0] = pltpu.async_copy(
            table_hbm.at[idx_v.at[pl.ds(0, chunk)]], bufs[0], sems[0])
        for c in range(nchunks):
            if c + 1 < nchunks:
                copies[c + 1] = pltpu.async_copy(
                    table_hbm.at[idx_v.at[pl.ds((c + 1) * chunk, chunk)]],
                    bufs[(c + 1) % 2], sems[(c + 1) % 2])
            copies[c].wait()
            pltpu.sync_copy(bufs[c % 2], out_hbm.at[pl.ds(base + c * chunk, chunk)])

    return gather_kernel(table, token_ids)


def _transform_pool(emb, cu_seqlens, W, b2, acc_in, offset, nseq, d, blk,
                    finalize):
    ntok = emb.shape[0]
    nblocks = ntok // blk

    def body(cu_ref, emb_ref, w_ref, b_ref, acc_ref, out_ref):
        i = pl.program_id(0)
        h = jnp.tanh(
            jnp.dot(emb_ref[...].astype(jnp.bfloat16),
                    w_ref[...].astype(jnp.bfloat16),
                    preferred_element_type=jnp.float32)
            + b_ref[...]
        )
        tok = jax.lax.broadcasted_iota(jnp.int32, (nseq, blk), 1) + offset + i * blk
        starts = jnp.concatenate(
            [jnp.full((1, blk), cu_ref[s], jnp.int32) for s in range(nseq)], axis=0)
        ends = jnp.concatenate(
            [jnp.full((1, blk), cu_ref[s + 1], jnp.int32) for s in range(nseq)], axis=0)
        onehot = ((tok >= starts) & (tok < ends)).astype(jnp.float32)
        partial = jnp.dot(onehot, h, preferred_element_type=jnp.float32)

        @pl.when(i == 0)
        def _init():
            out_ref[...] = acc_ref[...] + partial

        @pl.when(i > 0)
        def _acc():
            out_ref[...] += partial

        if finalize:
            @pl.when(i == nblocks - 1)
            def _finish():
                counts = jnp.concatenate(
                    [jnp.full((1, 1), cu_ref[s + 1] - cu_ref[s], jnp.int32)
                     for s in range(nseq)], axis=0)
                denom = jnp.maximum(counts.astype(jnp.float32), 1.0)
                out_ref[...] = out_ref[...] / denom

    return pl.pallas_call(
        body,
        grid=(nblocks,),
        in_specs=[
            pl.BlockSpec(memory_space=pltpu.SMEM),
            pl.BlockSpec((blk, d), lambda i: (i, 0)),
            pl.BlockSpec((d, d), lambda i: (0, 0)),
            pl.BlockSpec((1, d), lambda i: (0, 0)),
            pl.BlockSpec((nseq, d), lambda i: (0, 0)),
        ],
        out_specs=pl.BlockSpec((nseq, d), lambda i: (0, 0)),
        out_shape=jax.ShapeDtypeStruct((nseq, d), jnp.float32),
    )(cu_seqlens, emb, W, b2, acc_in)


def kernel(token_ids, cu_seqlens, table, W, b):
    total = token_ids.shape[0]
    d = table.shape[1]
    nseq = cu_seqlens.shape[0] - 1
    b2 = b.reshape(1, d)

    nsplit = 4
    half = total // nsplit
    embs = [
        _gather_rows(table, lax.slice(token_ids, (s * half,), ((s + 1) * half,)),
                     half, d)
        for s in range(nsplit)
    ]
    acc = jnp.zeros((nseq, d), jnp.float32)
    for s in range(nsplit):
        acc = _transform_pool(embs[s], cu_seqlens, W, b2, acc, offset=s * half,
                              nseq=nseq, d=d, blk=512,
                              finalize=(s == nsplit - 1))
    return acc


# speedup vs baseline: 1.1810x; 1.1810x over previous
import functools

import jax
import jax.numpy as jnp
from jax import lax
from jax.experimental import pallas as pl
from jax.experimental.pallas import tpu as pltpu
from jax.experimental.pallas import tpu_sc as plsc


def _gather_rows(table, token_ids, offset, ntok, d):
    info = plsc.get_sparse_core_info()
    nw = info.num_cores * info.num_subcores
    b_per_w = ntok // nw
    chunk = min(64, b_per_w)
    nchunks = b_per_w // chunk

    mesh = plsc.VectorSubcoreMesh(core_axis_name="c", subcore_axis_name="s")

    @functools.partial(
        pl.kernel,
        mesh=mesh,
        out_type=jax.ShapeDtypeStruct((ntok, d), jnp.float32),
        scratch_types=[
            pltpu.VMEM((b_per_w,), jnp.int32),
            pltpu.VMEM((chunk, d), jnp.float32),
            pltpu.VMEM((chunk, d), jnp.float32),
            pltpu.SemaphoreType.DMA,
            pltpu.SemaphoreType.DMA,
        ],
    )
    def gather_kernel(table_hbm, ids_hbm, out_hbm, idx_v, rows0, rows1, sem0, sem1):
        wid = lax.axis_index("s") * info.num_cores + lax.axis_index("c")
        base = wid * b_per_w
        pltpu.sync_copy(ids_hbm.at[pl.ds(offset + base, b_per_w)], idx_v)
        bufs = (rows0, rows1)
        sems = (sem0, sem1)
        copies = [None] * nchunks
        copies[0] = pltpu.async_copy(
            table_hbm.at[idx_v.at[pl.ds(0, chunk)]], bufs[0], sems[0])
        for c in range(nchunks):
            if c + 1 < nchunks:
                copies[c + 1] = pltpu.async_copy(
                    table_hbm.at[idx_v.at[pl.ds((c + 1) * chunk, chunk)]],
                    bufs[(c + 1) % 2], sems[(c + 1) % 2])
            copies[c].wait()
            pltpu.sync_copy(bufs[c % 2], out_hbm.at[pl.ds(base + c * chunk, chunk)])

    return gather_kernel(table, token_ids)


def _transform_pool(emb, cu_seqlens, Wbf, b2, acc_in, offset, nseq, d, blk,
                    finalize):
    ntok = emb.shape[0]
    nblocks = ntok // blk

    def body(cu_ref, emb_ref, w_ref, b_ref, acc_ref, out_ref):
        i = pl.program_id(0)
        h = jnp.tanh(
            jnp.dot(emb_ref[...].astype(jnp.bfloat16), w_ref[...],
                    preferred_element_type=jnp.float32)
            + b_ref[...]
        )
        tok = jax.lax.broadcasted_iota(jnp.int32, (nseq, blk), 1) + offset + i * blk
        starts = jnp.concatenate(
            [jnp.full((1, blk), cu_ref[s], jnp.int32) for s in range(nseq)], axis=0)
        ends = jnp.concatenate(
            [jnp.full((1, blk), cu_ref[s + 1], jnp.int32) for s in range(nseq)], axis=0)
        onehot = ((tok >= starts) & (tok < ends)).astype(jnp.bfloat16)
        partial = jnp.dot(onehot, h.astype(jnp.bfloat16),
                          preferred_element_type=jnp.float32)

        @pl.when(i == 0)
        def _init():
            out_ref[...] = acc_ref[...] + partial

        @pl.when(i > 0)
        def _acc():
            out_ref[...] += partial

        if finalize:
            @pl.when(i == nblocks - 1)
            def _finish():
                counts = jnp.concatenate(
                    [jnp.full((1, 1), cu_ref[s + 1] - cu_ref[s], jnp.int32)
                     for s in range(nseq)], axis=0)
                denom = jnp.maximum(counts.astype(jnp.float32), 1.0)
                out_ref[...] = out_ref[...] / denom

    return pl.pallas_call(
        body,
        grid=(nblocks,),
        in_specs=[
            pl.BlockSpec(memory_space=pltpu.SMEM),
            pl.BlockSpec((blk, d), lambda i: (i, 0)),
            pl.BlockSpec((d, d), lambda i: (0, 0)),
            pl.BlockSpec((1, d), lambda i: (0, 0)),
            pl.BlockSpec((nseq, d), lambda i: (0, 0)),
        ],
        out_specs=pl.BlockSpec((nseq, d), lambda i: (0, 0)),
        out_shape=jax.ShapeDtypeStruct((nseq, d), jnp.float32),
    )(cu_seqlens, emb, Wbf, b2, acc_in)


def kernel(token_ids, cu_seqlens, table, W, b):
    total = token_ids.shape[0]
    d = table.shape[1]
    nseq = cu_seqlens.shape[0] - 1
    b2 = b.reshape(1, d)
    Wbf = W.astype(jnp.bfloat16)

    nsplit = 2
    half = total // nsplit
    embs = [
        _gather_rows(table, token_ids, s * half, half, d)
        for s in range(nsplit)
    ]
    acc = jnp.zeros((nseq, d), jnp.float32)
    for s in range(nsplit):
        acc = _transform_pool(embs[s], cu_seqlens, Wbf, b2, acc, offset=s * half,
                              nseq=nseq, d=d, blk=1024,
                              finalize=(s == nsplit - 1))
    return acc
